# baseline (speedup 1.0000x reference)
import functools

import jax
import jax.numpy as jnp
from jax import lax
from jax.experimental import pallas as pl
from jax.experimental.pallas import tpu as pltpu
from jax.experimental.pallas import tpu_sc as plsc

_N_BOXES = 5000
_NW = 16
_CHUNK = 320
_N_PAD = _NW * _CHUNK
_NV = _CHUNK // 16
_IOU_THRESHOLD = 0.5
_MAX_OUT = 100
_OUT_PAD = 128
_IMG_SIZE = 512.0


def _sc_body(y1h, x1h, y2h, x2h, sh, outh,
             fy1, fx1, fy2, fx2,
             sy1, sx1, sy2, sx2, sarea, sms,
             lv, gred, sred, outv):
    wid = lax.axis_index("s")
    base = wid * _CHUNK

    pltpu.sync_copy(y1h, fy1)
    pltpu.sync_copy(x1h, fx1)
    pltpu.sync_copy(y2h, fy2)
    pltpu.sync_copy(x2h, fx2)
    pltpu.sync_copy(y1h.at[pl.ds(base, _CHUNK)], sy1)
    pltpu.sync_copy(x1h.at[pl.ds(base, _CHUNK)], sx1)
    pltpu.sync_copy(y2h.at[pl.ds(base, _CHUNK)], sy2)
    pltpu.sync_copy(x2h.at[pl.ds(base, _CHUNK)], sx2)
    pltpu.sync_copy(sh.at[pl.ds(base, _CHUNK)], sms)

    for i in range(_NV):
        sl = pl.ds(i * 16, 16)
        a = jnp.clip(sy1[sl], 0.0, _IMG_SIZE)
        b = jnp.clip(sx1[sl], 0.0, _IMG_SIZE)
        c = jnp.clip(sy2[sl], 0.0, _IMG_SIZE)
        d = jnp.clip(sx2[sl], 0.0, _IMG_SIZE)
        sy1[sl] = a
        sx1[sl] = b
        sy2[sl] = c
        sx2[sl] = d
        sarea[sl] = (c - a) * (d - b)

    zeros = jnp.zeros((16,), jnp.float32)
    for i in range(_OUT_PAD * 5 // 16):
        outv[pl.ds(i * 16, 16)] = zeros

    lane = lax.iota(jnp.int32, 16)

    def step(t, carry):
        vmax = sms[pl.ds(0, 16)]
        vidx = lane + base
        for i in range(1, _NV):
            v = sms[pl.ds(i * 16, 16)]
            iv = lane + (base + i * 16)
            upd = v > vmax
            vmax = jnp.where(upd, v, vmax)
            vidx = jnp.where(upd, iv, vidx)
        mloc = jnp.max(vmax)
        iloc = jnp.min(jnp.where(vmax == mloc, vidx, _N_PAD))

        lv[...] = jnp.where(lane == 0, mloc,
                            jnp.where(lane == 1, iloc.astype(jnp.float32), 0.0))
        pltpu.sync_copy(lv, sred.at[pl.ds(wid * 16, 16)])
        plsc.subcore_barrier()
        pltpu.sync_copy(sred, gred)
        plsc.subcore_barrier()

        vals = plsc.load_gather(gred, [lane * 16])
        idxs = plsc.load_gather(gred, [lane * 16 + 1])
        bv = jnp.max(vals)
        jsel = jnp.min(jnp.where(vals == bv, idxs,
                                 jnp.float32(_N_PAD))).astype(jnp.int32)

        @pl.when(bv > -0.5)
        def _():
            idxv = jnp.full((16,), jsel, jnp.int32)
            ey1 = plsc.load_gather(fy1, [idxv])[0]
            ex1 = plsc.load_gather(fx1, [idxv])[0]
            ey2 = plsc.load_gather(fy2, [idxv])[0]
            ex2 = plsc.load_gather(fx2, [idxv])[0]
            by1 = jnp.clip(ey1, 0.0, _IMG_SIZE)
            bx1 = jnp.clip(ex1, 0.0, _IMG_SIZE)
            by2 = jnp.clip(ey2, 0.0, _IMG_SIZE)
            bx2 = jnp.clip(ex2, 0.0, _IMG_SIZE)
            barea = (by2 - by1) * (bx2 - bx1)

            ovals = (jnp.where(lane == 0, by1, 0.0)
                     + jnp.where(lane == 1, bx1, 0.0)
                     + jnp.where(lane == 2, by2, 0.0)
                     + jnp.where(lane == 3, bx2, 0.0)
                     + jnp.where(lane == 4, bv, 0.0))
            oidx = t + lane * _OUT_PAD
            plsc.store_scatter(outv, [oidx], ovals, mask=lane < 5)

            for i in range(_NV):
                sl = pl.ds(i * 16, 16)
                iy1 = jnp.maximum(sy1[sl], by1)
                ix1 = jnp.maximum(sx1[sl], bx1)
                iy2 = jnp.minimum(sy2[sl], by2)
                ix2 = jnp.minimum(sx2[sl], bx2)
                inter = (jnp.maximum(iy2 - iy1, 0.0)
                         * jnp.maximum(ix2 - ix1, 0.0))
                union = sarea[sl] + barea - inter
                iou = inter / jnp.maximum(union, 1e-8)
                gix = lane + (base + i * 16)
                sms[sl] = jnp.where((iou > _IOU_THRESHOLD) | (gix == jsel),
                                    -1.0, sms[sl])

        return carry

    lax.fori_loop(0, _MAX_OUT, step, 0)

    @pl.when(wid == 0)
    def _():
        pltpu.sync_copy(outv, outh)


def _make_sc_call(interpret=False):
    mesh = plsc.VectorSubcoreMesh(core_axis_name="c", subcore_axis_name="s",
                                  num_cores=1, num_subcores=_NW)
    return pl.kernel(
        _sc_body,
        out_type=jax.ShapeDtypeStruct((_OUT_PAD * 5,), jnp.float32),
        mesh=mesh,
        scratch_types=[
            pltpu.VMEM((_N_PAD,), jnp.float32),
            pltpu.VMEM((_N_PAD,), jnp.float32),
            pltpu.VMEM((_N_PAD,), jnp.float32),
            pltpu.VMEM((_N_PAD,), jnp.float32),
            pltpu.VMEM((_CHUNK,), jnp.float32),
            pltpu.VMEM((_CHUNK,), jnp.float32),
            pltpu.VMEM((_CHUNK,), jnp.float32),
            pltpu.VMEM((_CHUNK,), jnp.float32),
            pltpu.VMEM((_CHUNK,), jnp.float32),
            pltpu.VMEM((_CHUNK,), jnp.float32),
            pltpu.VMEM((16,), jnp.float32),
            pltpu.VMEM((_NW * 16,), jnp.float32),
            pltpu.VMEM_SHARED((_NW * 16,), jnp.float32),
            pltpu.VMEM((_OUT_PAD * 5,), jnp.float32),
        ],
        compiler_params=pltpu.CompilerParams(needs_layout_passes=False),
        interpret=interpret,
    )


def kernel(boxes, scores):
    pad = _N_PAD - _N_BOXES
    y1 = jnp.pad(boxes[:, 0], (0, pad))
    x1 = jnp.pad(boxes[:, 1], (0, pad))
    y2 = jnp.pad(boxes[:, 2], (0, pad))
    x2 = jnp.pad(boxes[:, 3], (0, pad))
    s = jnp.pad(scores, (0, pad), constant_values=-1.0)
    outv = _make_sc_call()(y1, x1, y2, x2, s)
    return outv.reshape(5, _OUT_PAD).T[:_MAX_OUT]


# speedup vs baseline: 1.0898x; 1.0898x over previous
import jax
import jax.numpy as jnp
from jax import lax
from jax.experimental import pallas as pl
from jax.experimental.pallas import tpu as pltpu
from jax.experimental.pallas import tpu_sc as plsc

_N_BOXES = 5000
_NW = 16
_CHUNK = 320
_N_PAD = _NW * _CHUNK
_NV = _CHUNK // 16
_IOU_THRESHOLD = 0.5
_MAX_OUT = 100
_OUT_PAD = 128
_IMG_SIZE = 512.0
_RED = _NW * 16


def _sc_body(y1h, x1h, y2h, x2h, sh, outh,
             fy1, fx1, fy2, fx2,
             sy1, sx1, sy2, sx2, sarea, sms,
             lv, gred, sred, outv):
    wid = lax.axis_index("s")
    base = wid * _CHUNK

    pltpu.sync_copy(y1h, fy1)
    pltpu.sync_copy(x1h, fx1)
    pltpu.sync_copy(y2h, fy2)
    pltpu.sync_copy(x2h, fx2)
    pltpu.sync_copy(y1h.at[pl.ds(base, _CHUNK)], sy1)
    pltpu.sync_copy(x1h.at[pl.ds(base, _CHUNK)], sx1)
    pltpu.sync_copy(y2h.at[pl.ds(base, _CHUNK)], sy2)
    pltpu.sync_copy(x2h.at[pl.ds(base, _CHUNK)], sx2)
    pltpu.sync_copy(sh.at[pl.ds(base, _CHUNK)], sms)

    lane = lax.iota(jnp.int32, 16)

    vmax = jnp.full((16,), -1.0, jnp.float32)
    vidx = jnp.zeros((16,), jnp.int32)
    for i in range(_NV):
        sl = pl.ds(i * 16, 16)
        a = jnp.clip(sy1[sl], 0.0, _IMG_SIZE)
        b = jnp.clip(sx1[sl], 0.0, _IMG_SIZE)
        c = jnp.clip(sy2[sl], 0.0, _IMG_SIZE)
        d = jnp.clip(sx2[sl], 0.0, _IMG_SIZE)
        sy1[sl] = a
        sx1[sl] = b
        sy2[sl] = c
        sx2[sl] = d
        sarea[sl] = (c - a) * (d - b)
        v = sms[sl]
        upd = v > vmax
        vmax = jnp.where(upd, v, vmax)
        vidx = jnp.where(upd, lane + (base + i * 16), vidx)
    mloc0 = jnp.max(vmax)
    iloc0 = jnp.min(jnp.where(vmax == mloc0, vidx, _N_PAD))

    zeros = jnp.zeros((16,), jnp.float32)
    for i in range(_OUT_PAD * 5 // 16):
        outv[pl.ds(i * 16, 16)] = zeros

    def step(t, carry):
        mloc, iloc = carry
        lv[...] = jnp.where(lane == 0, mloc,
                            jnp.where(lane == 1, iloc.astype(jnp.float32), 0.0))
        off = (t % 2) * _RED
        pltpu.sync_copy(lv, sred.at[pl.ds(off + wid * 16, 16)])
        plsc.subcore_barrier()
        pltpu.sync_copy(sred.at[pl.ds(off, _RED)], gred)

        vals = plsc.load_gather(gred, [lane * 16])
        idxs = plsc.load_gather(gred, [lane * 16 + 1])
        bv = jnp.max(vals)
        jsel = jnp.min(jnp.where(vals == bv, idxs,
                                 jnp.float32(_N_PAD))).astype(jnp.int32)

        def do():
            idxv = jnp.full((16,), jsel, jnp.int32)
            ey1 = plsc.load_gather(fy1, [idxv])[0]
            ex1 = plsc.load_gather(fx1, [idxv])[0]
            ey2 = plsc.load_gather(fy2, [idxv])[0]
            ex2 = plsc.load_gather(fx2, [idxv])[0]
            by1 = jnp.clip(ey1, 0.0, _IMG_SIZE)
            bx1 = jnp.clip(ex1, 0.0, _IMG_SIZE)
            by2 = jnp.clip(ey2, 0.0, _IMG_SIZE)
            bx2 = jnp.clip(ex2, 0.0, _IMG_SIZE)
            barea = (by2 - by1) * (bx2 - bx1)

            ovals = (jnp.where(lane == 0, by1, 0.0)
                     + jnp.where(lane == 1, bx1, 0.0)
                     + jnp.where(lane == 2, by2, 0.0)
                     + jnp.where(lane == 3, bx2, 0.0)
                     + jnp.where(lane == 4, bv, 0.0))
            oidx = t + lane * _OUT_PAD
            plsc.store_scatter(outv, [oidx], ovals, mask=lane < 5)

            vmax = jnp.full((16,), -1.0, jnp.float32)
            vidx = jnp.zeros((16,), jnp.int32)
            for i in range(_NV):
                sl = pl.ds(i * 16, 16)
                iy1 = jnp.maximum(sy1[sl], by1)
                ix1 = jnp.maximum(sx1[sl], bx1)
                iy2 = jnp.minimum(sy2[sl], by2)
                ix2 = jnp.minimum(sx2[sl], bx2)
                inter = (jnp.maximum(iy2 - iy1, 0.0)
                         * jnp.maximum(ix2 - ix1, 0.0))
                union = sarea[sl] + barea - inter
                iou = inter / jnp.maximum(union, 1e-8)
                gix = lane + (base + i * 16)
                msn = jnp.where((iou > _IOU_THRESHOLD) | (gix == jsel),
                                -1.0, sms[sl])
                sms[sl] = msn
                upd = msn > vmax
                vmax = jnp.where(upd, msn, vmax)
                vidx = jnp.where(upd, gix, vidx)
            mloc2 = jnp.max(vmax)
            iloc2 = jnp.min(jnp.where(vmax == mloc2, vidx, _N_PAD))
            return mloc2, iloc2

        return lax.cond(bv > -0.5, do, lambda: (mloc, iloc))

    lax.fori_loop(0, _MAX_OUT, step, (mloc0, iloc0))

    @pl.when(wid == 0)
    def _():
        pltpu.sync_copy(outv, outh)


def _make_sc_call(interpret=False):
    mesh = plsc.VectorSubcoreMesh(core_axis_name="c", subcore_axis_name="s",
                                  num_cores=1, num_subcores=_NW)
    return pl.kernel(
        _sc_body,
        out_type=jax.ShapeDtypeStruct((_OUT_PAD * 5,), jnp.float32),
        mesh=mesh,
        scratch_types=[
            pltpu.VMEM((_N_PAD,), jnp.float32),
            pltpu.VMEM((_N_PAD,), jnp.float32),
            pltpu.VMEM((_N_PAD,), jnp.float32),
            pltpu.VMEM((_N_PAD,), jnp.float32),
            pltpu.VMEM((_CHUNK,), jnp.float32),
            pltpu.VMEM((_CHUNK,), jnp.float32),
            pltpu.VMEM((_CHUNK,), jnp.float32),
            pltpu.VMEM((_CHUNK,), jnp.float32),
            pltpu.VMEM((_CHUNK,), jnp.float32),
            pltpu.VMEM((_CHUNK,), jnp.float32),
            pltpu.VMEM((16,), jnp.float32),
            pltpu.VMEM((_RED,), jnp.float32),
            pltpu.VMEM_SHARED((2 * _RED,), jnp.float32),
            pltpu.VMEM((_OUT_PAD * 5,), jnp.float32),
        ],
        compiler_params=pltpu.CompilerParams(needs_layout_passes=False),
        interpret=interpret,
    )


def kernel(boxes, scores):
    pad = _N_PAD - _N_BOXES
    y1 = jnp.pad(boxes[:, 0], (0, pad))
    x1 = jnp.pad(boxes[:, 1], (0, pad))
    y2 = jnp.pad(boxes[:, 2], (0, pad))
    x2 = jnp.pad(boxes[:, 3], (0, pad))
    s = jnp.pad(scores, (0, pad), constant_values=-1.0)
    outv = _make_sc_call()(y1, x1, y2, x2, s)
    return outv.reshape(5, _OUT_PAD).T[:_MAX_OUT]


# speedup vs baseline: 1.2658x; 1.1616x over previous
import jax
import jax.numpy as jnp
from jax import lax
from jax.experimental import pallas as pl
from jax.experimental.pallas import tpu as pltpu
from jax.experimental.pallas import tpu_sc as plsc

_N_BOXES = 5000
_NW = 16
_CHUNK = 320
_N_PAD = _NW * _CHUNK
_NV = _CHUNK // 16
_IOU_THRESHOLD = 0.5
_MAX_OUT = 100
_OUT_PAD = 128
_IMG_SIZE = 512.0
_RED = _NW * 16
_BIG = float(_N_PAD)


def _sc_body(y1h, x1h, y2h, x2h, sh, outh,
             fy1, fx1, fy2, fx2,
             sy1, sx1, sy2, sx2, sarea, sms,
             lv, gred, sred, outv):
    wid = lax.axis_index("s")
    base = wid * _CHUNK

    pltpu.sync_copy(y1h, fy1)
    pltpu.sync_copy(x1h, fx1)
    pltpu.sync_copy(y2h, fy2)
    pltpu.sync_copy(x2h, fx2)
    pltpu.sync_copy(y1h.at[pl.ds(base, _CHUNK)], sy1)
    pltpu.sync_copy(x1h.at[pl.ds(base, _CHUNK)], sx1)
    pltpu.sync_copy(y2h.at[pl.ds(base, _CHUNK)], sy2)
    pltpu.sync_copy(x2h.at[pl.ds(base, _CHUNK)], sx2)
    pltpu.sync_copy(sh.at[pl.ds(base, _CHUNK)], sms)

    lane = lax.iota(jnp.int32, 16)

    def top2_insert(v, gix, v1, i1, v2, i2):
        upd1 = v > v1
        upd2 = v > v2
        v2n = jnp.where(upd1, v1, jnp.where(upd2, v, v2))
        i2n = jnp.where(upd1, i1, jnp.where(upd2, gix, i2))
        v1n = jnp.where(upd1, v, v1)
        i1n = jnp.where(upd1, gix, i1)
        return v1n, i1n, v2n, i2n

    def top2_scalarize(v1, i1, v2, i2):
        i1f = i1.astype(jnp.float32)
        i2f = i2.astype(jnp.float32)
        m1 = jnp.max(v1)
        s1 = jnp.min(jnp.where(v1 == m1, i1f, _BIG))
        c2v = jnp.where(i1f == s1, -1.0, v1)
        m2 = jnp.maximum(jnp.max(c2v), jnp.max(v2))
        s2 = jnp.minimum(jnp.min(jnp.where(c2v == m2, i1f, _BIG)),
                         jnp.min(jnp.where(v2 == m2, i2f, _BIG)))
        return m1, s1, m2, s2

    v1 = jnp.full((16,), -1.0, jnp.float32)
    i1 = jnp.zeros((16,), jnp.int32)
    v2 = jnp.full((16,), -1.0, jnp.float32)
    i2 = jnp.zeros((16,), jnp.int32)
    for i in range(_NV):
        sl = pl.ds(i * 16, 16)
        a = jnp.clip(sy1[sl], 0.0, _IMG_SIZE)
        b = jnp.clip(sx1[sl], 0.0, _IMG_SIZE)
        c = jnp.clip(sy2[sl], 0.0, _IMG_SIZE)
        d = jnp.clip(sx2[sl], 0.0, _IMG_SIZE)
        sy1[sl] = a
        sx1[sl] = b
        sy2[sl] = c
        sx2[sl] = d
        sarea[sl] = (c - a) * (d - b)
        v1, i1, v2, i2 = top2_insert(sms[sl], lane + (base + i * 16),
                                     v1, i1, v2, i2)
    m1, s1, m2, s2 = top2_scalarize(v1, i1, v2, i2)

    zeros = jnp.zeros((16,), jnp.float32)
    for i in range(_OUT_PAD * 5 // 16):
        outv[pl.ds(i * 16, 16)] = zeros

    def get_box(jsel):
        idxv = jnp.full((16,), jsel, jnp.int32)
        ey1 = plsc.load_gather(fy1, [idxv])[0]
        ex1 = plsc.load_gather(fx1, [idxv])[0]
        ey2 = plsc.load_gather(fy2, [idxv])[0]
        ex2 = plsc.load_gather(fx2, [idxv])[0]
        by1 = jnp.clip(ey1, 0.0, _IMG_SIZE)
        bx1 = jnp.clip(ex1, 0.0, _IMG_SIZE)
        by2 = jnp.clip(ey2, 0.0, _IMG_SIZE)
        bx2 = jnp.clip(ex2, 0.0, _IMG_SIZE)
        return by1, bx1, by2, bx2, (by2 - by1) * (bx2 - bx1)

    def cond_fn(carry):
        it, kcnt, done, m1, s1, m2, s2 = carry
        return (kcnt < _MAX_OUT) & (done == 0)

    def body_fn(carry):
        it, kcnt, done, m1, s1, m2, s2 = carry
        lv[...] = jnp.where(lane == 0, m1,
                  jnp.where(lane == 1, s1,
                  jnp.where(lane == 2, m2,
                  jnp.where(lane == 3, s2, 0.0))))
        off = (it % 2) * _RED
        pltpu.sync_copy(lv, sred.at[pl.ds(off + wid * 16, 16)])
        plsc.subcore_barrier()
        pltpu.sync_copy(sred.at[pl.ds(off, _RED)], gred)

        g1v = plsc.load_gather(gred, [lane * 16])
        g1i = plsc.load_gather(gred, [lane * 16 + 1])
        g2v = plsc.load_gather(gred, [lane * 16 + 2])
        g2i = plsc.load_gather(gred, [lane * 16 + 3])
        bv1 = jnp.max(g1v)
        js1f = jnp.min(jnp.where(g1v == bv1, g1i, _BIG))
        c2v = jnp.where(g1i == js1f, -1.0, g1v)
        bv2 = jnp.maximum(jnp.max(c2v), jnp.max(g2v))
        js2f = jnp.minimum(jnp.min(jnp.where(c2v == bv2, g1i, _BIG)),
                           jnp.min(jnp.where(g2v == bv2, g2i, _BIG)))

        def do():
            jsel1 = js1f.astype(jnp.int32)
            jsel2 = jnp.minimum(js2f,
                                jnp.float32(_N_PAD - 1)).astype(jnp.int32)
            ay1, ax1, ay2, ax2, aarea = get_box(jsel1)
            by1, bx1, by2, bx2, barea = get_box(jsel2)
            wiy1 = jnp.maximum(ay1, by1)
            wix1 = jnp.maximum(ax1, bx1)
            wiy2 = jnp.minimum(ay2, by2)
            wix2 = jnp.minimum(ax2, bx2)
            winter = (jnp.maximum(wiy2 - wiy1, 0.0)
                      * jnp.maximum(wix2 - wix1, 0.0))
            wunion = aarea + barea - winter
            wiou = (jnp.full((16,), winter)
                    / jnp.full((16,), jnp.maximum(wunion, 1e-8)))[0]
            both = ((bv2 > -0.5) & (kcnt < _MAX_OUT - 1)
                    & jnp.logical_not(wiou > _IOU_THRESHOLD))

            oval1 = (jnp.where(lane == 0, ay1, 0.0)
                     + jnp.where(lane == 1, ax1, 0.0)
                     + jnp.where(lane == 2, ay2, 0.0)
                     + jnp.where(lane == 3, ax2, 0.0)
                     + jnp.where(lane == 4, bv1, 0.0))
            plsc.store_scatter(outv, [kcnt + lane * _OUT_PAD], oval1,
                               mask=lane < 5)
            oval2 = (jnp.where(lane == 0, by1, 0.0)
                     + jnp.where(lane == 1, bx1, 0.0)
                     + jnp.where(lane == 2, by2, 0.0)
                     + jnp.where(lane == 3, bx2, 0.0)
                     + jnp.where(lane == 4, bv2, 0.0))
            plsc.store_scatter(outv, [kcnt + 1 + lane * _OUT_PAD], oval2,
                               mask=(lane < 5) & both)

            v1 = jnp.full((16,), -1.0, jnp.float32)
            i1 = jnp.zeros((16,), jnp.int32)
            v2 = jnp.full((16,), -1.0, jnp.float32)
            i2 = jnp.zeros((16,), jnp.int32)
            for i in range(_NV):
                sl = pl.ds(i * 16, 16)
                ty1 = sy1[sl]
                tx1 = sx1[sl]
                ty2 = sy2[sl]
                tx2 = sx2[sl]
                tarea = sarea[sl]
                iy1 = jnp.maximum(ty1, ay1)
                ix1 = jnp.maximum(tx1, ax1)
                iy2 = jnp.minimum(ty2, ay2)
                ix2 = jnp.minimum(tx2, ax2)
                inter1 = (jnp.maximum(iy2 - iy1, 0.0)
                          * jnp.maximum(ix2 - ix1, 0.0))
                iou1 = inter1 / jnp.maximum(tarea + aarea - inter1, 1e-8)
                jy1 = jnp.maximum(ty1, by1)
                jx1 = jnp.maximum(tx1, bx1)
                jy2 = jnp.minimum(ty2, by2)
                jx2 = jnp.minimum(tx2, bx2)
                inter2 = (jnp.maximum(jy2 - jy1, 0.0)
                          * jnp.maximum(jx2 - jx1, 0.0))
                iou2 = inter2 / jnp.maximum(tarea + barea - inter2, 1e-8)
                gix = lane + (base + i * 16)
                sup = ((iou1 > _IOU_THRESHOLD) | (gix == jsel1)
                       | (both & ((iou2 > _IOU_THRESHOLD) | (gix == jsel2))))
                msn = jnp.where(sup, -1.0, sms[sl])
                sms[sl] = msn
                v1, i1, v2, i2 = top2_insert(msn, gix, v1, i1, v2, i2)
            m1n, s1n, m2n, s2n = top2_scalarize(v1, i1, v2, i2)
            kn = kcnt + 1 + jnp.where(both, 1, 0)
            return kn, jnp.bool_(False), m1n, s1n, m2n, s2n

        def skip():
            return kcnt, jnp.bool_(True), m1, s1, m2, s2

        kn, done_n, m1n, s1n, m2n, s2n = lax.cond(bv1 > -0.5, do, skip)
        return it + 1, kn, done_n, m1n, s1n, m2n, s2n

    lax.while_loop(cond_fn, body_fn,
                   (jnp.int32(0), jnp.int32(0), jnp.bool_(False),
                    m1, s1, m2, s2))

    @pl.when(wid == 0)
    def _():
        pltpu.sync_copy(outv, outh)


def _make_sc_call(interpret=False):
    mesh = plsc.VectorSubcoreMesh(core_axis_name="c", subcore_axis_name="s",
                                  num_cores=1, num_subcores=_NW)
    return pl.kernel(
        _sc_body,
        out_type=jax.ShapeDtypeStruct((_OUT_PAD * 5,), jnp.float32),
        mesh=mesh,
        scratch_types=[
            pltpu.VMEM((_N_PAD,), jnp.float32),
            pltpu.VMEM((_N_PAD,), jnp.float32),
            pltpu.VMEM((_N_PAD,), jnp.float32),
            pltpu.VMEM((_N_PAD,), jnp.float32),
            pltpu.VMEM((_CHUNK,), jnp.float32),
            pltpu.VMEM((_CHUNK,), jnp.float32),
            pltpu.VMEM((_CHUNK,), jnp.float32),
            pltpu.VMEM((_CHUNK,), jnp.float32),
            pltpu.VMEM((_CHUNK,), jnp.float32),
            pltpu.VMEM((_CHUNK,), jnp.float32),
            pltpu.VMEM((16,), jnp.float32),
            pltpu.VMEM((_RED,), jnp.float32),
            pltpu.VMEM_SHARED((2 * _RED,), jnp.float32),
            pltpu.VMEM((_OUT_PAD * 5,), jnp.float32),
        ],
        compiler_params=pltpu.CompilerParams(needs_layout_passes=False),
        interpret=interpret,
    )


def kernel(boxes, scores):
    pad = _N_PAD - _N_BOXES
    y1 = jnp.pad(boxes[:, 0], (0, pad))
    x1 = jnp.pad(boxes[:, 1], (0, pad))
    y2 = jnp.pad(boxes[:, 2], (0, pad))
    x2 = jnp.pad(boxes[:, 3], (0, pad))
    s = jnp.pad(scores, (0, pad), constant_values=-1.0)
    outv = _make_sc_call()(y1, x1, y2, x2, s)
    return outv.reshape(5, _OUT_PAD).T[:_MAX_OUT]
